# baseline (speedup 1.0000x reference)
import functools

import jax
import jax.numpy as jnp
from jax import lax
from jax.experimental import pallas as pl
from jax.experimental.pallas import tpu as pltpu
from jax.experimental.pallas import tpu_sc as plsc

_B = 16384
_K = 16
_ROWS_PER_LINE = 8
_NC = 2
_NS = 16
_NW = _NC * _NS
_BPW = _B // _NW
_CHUNK = 128
_NCHUNK = _BPW // _CHUNK
_L = 16


def _mf_body(w_hbm, h_hbm, ulines_hbm, ilines_hbm, uoffs_hbm, ioffs_hbm,
             out_hbm, ul_v, il_v, uo_v, io_v, ubuf, vbuf, out_v, sem):
    wid = lax.axis_index("s") * _NC + lax.axis_index("c")

    pltpu.sync_copy(ulines_hbm.at[wid], ul_v)
    pltpu.sync_copy(ilines_hbm.at[wid], il_v)
    pltpu.sync_copy(uoffs_hbm.at[wid], uo_v)
    pltpu.sync_copy(ioffs_hbm.at[wid], io_v)

    for j in range(_NCHUNK):
        cu = pltpu.async_copy(w_hbm.at[ul_v.at[j]], ubuf, sem)
        cv = pltpu.async_copy(h_hbm.at[il_v.at[j]], vbuf, sem)
        cu.wait()
        cv.wait()

        def _dot16(c, carry, j=j):
            rows = c * _L + lax.iota(jnp.int32, _L)
            ucol = uo_v[j, pl.ds(c * _L, _L)]
            icol = io_v[j, pl.ds(c * _L, _L)]
            acc = jnp.zeros((_L,), jnp.float32)
            for k in range(_K):
                u = plsc.load_gather(ubuf, [rows, ucol + k])
                v = plsc.load_gather(vbuf, [rows, icol + k])
                acc = acc + u * v
            out_v[pl.ds(j * _CHUNK + c * _L, _L)] = 1.0 / (1.0 + jnp.exp(-acc))
            return carry

        lax.fori_loop(0, _CHUNK // _L, _dot16, 0)

    pltpu.sync_copy(out_v, out_hbm.at[pl.ds(wid * _BPW, _BPW)])


_mf_call = functools.partial(
    pl.kernel,
    out_type=jax.ShapeDtypeStruct((_B,), jnp.float32),
    mesh=plsc.VectorSubcoreMesh(core_axis_name="c", subcore_axis_name="s"),
    scratch_types=[
        pltpu.VMEM((_NCHUNK, _CHUNK), jnp.int32),
        pltpu.VMEM((_NCHUNK, _CHUNK), jnp.int32),
        pltpu.VMEM((_NCHUNK, _CHUNK), jnp.int32),
        pltpu.VMEM((_NCHUNK, _CHUNK), jnp.int32),
        pltpu.VMEM((_CHUNK, 8 * _K), jnp.float32),
        pltpu.VMEM((_CHUNK, 8 * _K), jnp.float32),
        pltpu.VMEM((_BPW,), jnp.float32),
        pltpu.SemaphoreType.DMA,
    ],
    compiler_params=pltpu.CompilerParams(
        needs_layout_passes=False, use_tc_tiling_on_sc=False),
)(_mf_body)


def kernel(x, W, H):
    wl = W.reshape(-1, _ROWS_PER_LINE * _K)
    hl = H.reshape(-1, _ROWS_PER_LINE * _K)
    uidx = x[:, 0]
    iidx = x[:, 1]
    shape = (_NW, _NCHUNK, _CHUNK)
    ulines = (uidx // _ROWS_PER_LINE).reshape(shape)
    ilines = (iidx // _ROWS_PER_LINE).reshape(shape)
    uoffs = ((uidx % _ROWS_PER_LINE) * _K).reshape(shape)
    ioffs = ((iidx % _ROWS_PER_LINE) * _K).reshape(shape)
    return _mf_call(wl, hl, ulines, ilines, uoffs, ioffs)
